# Initial kernel scaffold; baseline (speedup 1.0000x reference)
import jax
import jax.numpy as jnp
from jax.experimental import pallas as pl

KMAX = 512
OFFSETS = [(dx, dy, dz) for dx in (-1, 0, 1) for dy in (-1, 0, 1) for dz in (-1, 0, 1)]


def _key(c):
    c = c.astype(jnp.int64)
    return ((c[:, 0] * KMAX + c[:, 1]) * KMAX + c[:, 2]) * KMAX + c[:, 3]


def _lookup(sorted_keys, order, qk, valid):
    pos = jnp.clip(jnp.searchsorted(sorted_keys, qk), 0, sorted_keys.shape[0] - 1)
    found = (sorted_keys[pos] == qk) & valid
    return order[pos], found


def _downsample(coords, stride):
    if stride == 1:
        return coords
    d = jnp.concatenate([coords[:, :1], coords[:, 1:] // stride], axis=1)
    return jnp.unique(d, axis=0, size=d.shape[0], fill_value=-1)


def _sparse_conv(in_coords, in_feats, W, stride, out_coords):
    keys = _key(in_coords)
    order = jnp.argsort(keys)
    sk = keys[order]
    out = jnp.zeros((out_coords.shape[0], W.shape[2]), in_feats.dtype)
    base = out_coords[:, 1:] * stride
    for k, off in enumerate(OFFSETS):
        nb = base + jnp.array(off, out_coords.dtype)
        valid = jnp.all((nb >= 0) & (nb < KMAX), axis=1)
        qc = jnp.concatenate([out_coords[:, :1], nb], axis=1)
        idx, found = _lookup(sk, order, _key(qc), valid)
        out = out + jnp.where(found[:, None], in_feats[idx], 0.0) @ W[k]
    return out


def _sparse_conv_tr(in_coords, in_feats, W, stride, out_coords):
    keys = _key(in_coords)
    order = jnp.argsort(keys)
    sk = keys[order]
    out = jnp.zeros((out_coords.shape[0], W.shape[2]), in_feats.dtype)
    for k, off in enumerate(OFFSETS):
        cand = out_coords[:, 1:] - jnp.array(off, out_coords.dtype)
        p = cand // stride
        ok = jnp.all(cand % stride == 0, axis=1) & jnp.all((cand >= 0) & (p < KMAX), axis=1)
        qc = jnp.concatenate([out_coords[:, :1], p], axis=1)
        idx, found = _lookup(sk, order, _key(qc), ok)
        out = out + jnp.where(found[:, None], in_feats[idx], 0.0) @ W[k]
    return out


def _bn_relu(x, g, b, relu=True, mask=None):
    if mask is None:
        y = (x - x.mean(axis=0)) / jnp.sqrt(x.var(axis=0) + 1e-5) * g + b
    else:
        m = mask.astype(x.dtype)[:, None]
        n = jnp.sum(m)
        mu = jnp.sum(x * m, axis=0) / n
        var = jnp.sum(((x - mu) ** 2) * m, axis=0) / n
        y = (x - mu) / jnp.sqrt(var + 1e-5) * g + b
    return jax.nn.relu(y) if relu else y


def _copy_kernel(x_ref, o_ref):
    o_ref[...] = x_ref[...]


def kernel(coords, feats, params):
    c1 = coords
    c2 = _downsample(c1, 2)
    c4 = _downsample(c2, 2)
    c8 = _downsample(c4, 2)
    m2 = c2[:, 0] >= 0
    m4 = c4[:, 0] >= 0
    m8 = c8[:, 0] >= 0
    p = params
    f1 = _bn_relu(_sparse_conv(c1, feats, p['W0'], 1, c1), p['g0'], p['b0'])
    f2 = _bn_relu(_sparse_conv(c1, f1, p['W1'], 2, c2), p['g1'], p['b1'], mask=m2)
    f4 = _bn_relu(_sparse_conv(c2, f2, p['W2'], 2, c4), p['g2'], p['b2'], mask=m4)
    f8 = _bn_relu(_sparse_conv(c4, f4, p['W3'], 2, c8), p['g3'], p['b3'], mask=m8)
    d = _bn_relu(_sparse_conv_tr(c8, f8, p['Wt3'], 2, c4), p['g4'], p['b4'], mask=m4)
    d = jnp.concatenate([d, f4], axis=1)
    d = _bn_relu(_sparse_conv_tr(c4, d, p['Wt2'], 2, c2), p['g5'], p['b5'], mask=m2)
    d = jnp.concatenate([d, f2], axis=1)
    d = _bn_relu(_sparse_conv_tr(c2, d, p['Wt1'], 2, c1), p['g6'], p['b6'])
    d = jnp.concatenate([d, f1], axis=1)
    out = _sparse_conv_tr(c1, d, p['Wt0'], 1, c1)
    return pl.pallas_call(
        _copy_kernel,
        out_shape=jax.ShapeDtypeStruct(out.shape, out.dtype),
    )(out)


# speedup vs baseline: 1.9427x; 1.9427x over previous
import functools

import jax
import jax.numpy as jnp
from jax import lax
from jax.experimental import pallas as pl
from jax.experimental.pallas import tpu as pltpu
from jax.experimental.pallas import tpu_sc as plsc

OFFS = [(dx, dy, dz) for dx in (-1, 0, 1) for dy in (-1, 0, 1) for dz in (-1, 0, 1)]
NW = 32
NT1 = 16
GPAD = 32768
Bm = 2048

_SCP = pltpu.CompilerParams(use_tc_tiling_on_sc=False)


def _mesh(nc):
    return plsc.VectorSubcoreMesh(core_axis_name="c", subcore_axis_name="s", num_cores=nc)


def _wid2():
    return lax.axis_index("s") * 2 + lax.axis_index("c")


def _make_build(Np, CH16, specs, masked):
    nout = len(specs)

    def body(*refs):
        px, py, pz = refs[0:3]
        r = 3
        mk = None
        if masked:
            mk = refs[r]; r += 1
        grids = refs[r:r + nout]; r += nout
        cx, cy, cz, kb, vb = refs[r:r + 5]
        sem = refs[r + 5]
        tid = lax.axis_index("s")
        nv = CH16 // 16

        def fset(i, _):
            vb[pl.ds(i * 16, 16)] = jnp.full((16,), Np, jnp.int32)
            return 0
        lax.fori_loop(0, 2048 // 16, fset, 0)
        for gi, (S, sh) in enumerate(specs):
            G = S * S * S + GPAD
            slc = G // NT1
            base = tid * slc
            gref = grids[gi]

            def fcopy(j, _, gref=gref, base=base):
                pltpu.sync_copy(vb.at[pl.ds(0, 2048)], gref.at[pl.ds(base + j * 2048, 2048)])
                return 0
            lax.fori_loop(0, slc // 2048, fcopy, 0)
        plsc.subcore_barrier()

        rbase = tid * CH16
        pltpu.sync_copy(px.at[pl.ds(rbase, CH16)], cx)
        pltpu.sync_copy(py.at[pl.ds(rbase, CH16)], cy)
        pltpu.sync_copy(pz.at[pl.ds(rbase, CH16)], cz)
        if masked:
            pltpu.sync_copy(mk.at[pl.ds(rbase, CH16)], vb)
        for gi, (S, sh) in enumerate(specs):
            gref = grids[gi]
            DUMP = S * S * S

            def kstep(i, _, S=S, sh=sh, DUMP=DUMP):
                x = cx[pl.ds(i * 16, 16)] >> sh
                y = cy[pl.ds(i * 16, 16)] >> sh
                z = cz[pl.ds(i * 16, 16)] >> sh
                key = (x * S + y) * S + z
                key = jnp.minimum(key, jnp.full((16,), DUMP, jnp.int32))
                if masked:
                    m = vb[pl.ds(i * 16, 16)]
                    key = jnp.where(m > 0, key, jnp.full((16,), DUMP, jnp.int32))
                kb[pl.ds(i * 16, 16)] = key
                return 0
            lax.fori_loop(0, nv, kstep, 0)

            def istep(i, _):
                vb[pl.ds(i * 16, 16)] = (jnp.full((16,), rbase + i * 16, jnp.int32)
                                         + lax.iota(jnp.int32, 16))
                return 0
            lax.fori_loop(0, nv, istep, 0)
            pltpu.sync_copy(vb.at[pl.ds(0, CH16)], gref.at[kb])

    return body


def _build_grids(px, py, pz, Np, CH16, specs, masked_arr=None):
    outs = [jax.ShapeDtypeStruct((S * S * S + GPAD,), jnp.int32) for S, _ in specs]
    scratch = [pltpu.VMEM((CH16,), jnp.int32) for _ in range(5)] + [pltpu.SemaphoreType.DMA]
    body = _make_build(Np, CH16, specs, masked_arr is not None)
    args = (px, py, pz) + ((masked_arr,) if masked_arr is not None else ())
    fn = functools.partial(
        pl.kernel, out_type=tuple(outs) if len(outs) > 1 else outs[0],
        mesh=_mesh(1), scratch_types=scratch, compiler_params=_SCP,
    )(body)
    return fn(*args)


def _make_rep(Np, CH, S, sh, N, has_parent):
    DUMP = S * S * S

    def body(*refs):
        px, py, pz = refs[0:3]
        r = 3
        par = None
        if has_parent:
            par = refs[r]; r += 1
        grid = refs[r]; rep = refs[r + 1]
        cx, cy, cz, kb, gb = refs[r + 2:r + 7]
        sem = refs[r + 7]
        wid = _wid2()
        rbase = wid * CH
        nv = CH // 16
        pltpu.sync_copy(px.at[pl.ds(rbase, CH)], cx)
        pltpu.sync_copy(py.at[pl.ds(rbase, CH)], cy)
        pltpu.sync_copy(pz.at[pl.ds(rbase, CH)], cz)
        if has_parent:
            pltpu.sync_copy(par.at[pl.ds(rbase, CH)], gb)

        def kstep(i, _):
            x = cx[pl.ds(i * 16, 16)] >> sh
            y = cy[pl.ds(i * 16, 16)] >> sh
            z = cz[pl.ds(i * 16, 16)] >> sh
            key = (x * S + y) * S + z
            kb[pl.ds(i * 16, 16)] = jnp.minimum(key, jnp.full((16,), DUMP, jnp.int32))
            return 0
        lax.fori_loop(0, nv, kstep, 0)
        pltpu.async_copy(grid.at[kb], cx, sem).wait()

        def fix(i, _):
            g = cx[pl.ds(i * 16, 16)]
            one = jnp.full((16,), 1, jnp.int32)
            zero = jnp.full((16,), 0, jnp.int32)
            rid = jnp.full((16,), rbase + i * 16, jnp.int32) + lax.iota(jnp.int32, 16)
            ok = jnp.where(g == rid, one, zero)
            ok = ok * jnp.where(rid < jnp.full((16,), N, jnp.int32), one, zero)
            if has_parent:
                ok = ok * jnp.where(gb[pl.ds(i * 16, 16)] > 0, one, zero)
            cy[pl.ds(i * 16, 16)] = ok
            return 0
        lax.fori_loop(0, nv, fix, 0)
        pltpu.sync_copy(cy.at[pl.ds(0, CH)], rep.at[pl.ds(rbase, CH)])

    return body


def _rep(px, py, pz, grid, Np, CH, S, sh, N, parent=None):
    scratch = [pltpu.VMEM((CH,), jnp.int32) for _ in range(5)] + [pltpu.SemaphoreType.DMA]
    body = _make_rep(Np, CH, S, sh, N, parent is not None)
    args = (px, py, pz) + ((parent,) if parent is not None else ()) + (grid,)
    return functools.partial(
        pl.kernel, out_type=jax.ShapeDtypeStruct((Np + Bm,), jnp.int32),
        mesh=_mesh(2), scratch_types=scratch, compiler_params=_SCP,
    )(body)(*args)


def _make_kq(Np, CH, S, sh, stride, transposed, mode, SENT):
    DUMP = S * S * S

    def body(*refs):
        px, py, pz, grid = refs[0:4]
        nout = 2 if mode == "both" else 1
        outs = refs[4:4 + nout]
        cx, cy, cz, kb, ob, gb = refs[4 + nout:10 + nout]
        sem = refs[10 + nout]
        wid = _wid2()
        rbase = wid * CH
        nv = CH // 16
        pltpu.sync_copy(px.at[pl.ds(rbase, CH)], cx)
        pltpu.sync_copy(py.at[pl.ds(rbase, CH)], cy)
        pltpu.sync_copy(pz.at[pl.ds(rbase, CH)], cz)
        if sh:
            def shl(i, _):
                cx[pl.ds(i * 16, 16)] = cx[pl.ds(i * 16, 16)] >> sh
                cy[pl.ds(i * 16, 16)] = cy[pl.ds(i * 16, 16)] >> sh
                cz[pl.ds(i * 16, 16)] = cz[pl.ds(i * 16, 16)] >> sh
                return 0
            lax.fori_loop(0, nv, shl, 0)

        for k, (ox, oy, oz) in enumerate(OFFS):
            def qstep(i, _, ox=ox, oy=oy, oz=oz):
                x = cx[pl.ds(i * 16, 16)]
                y = cy[pl.ds(i * 16, 16)]
                z = cz[pl.ds(i * 16, 16)]
                one = jnp.full((16,), 1, jnp.int32)
                zero = jnp.full((16,), 0, jnp.int32)
                Sv = jnp.full((16,), S, jnp.int32)
                if not transposed:
                    nx = x * stride + ox
                    ny = y * stride + oy
                    nz = z * stride + oz
                    ok = jnp.where(nx >= zero, one, zero)
                    ok = ok * jnp.where(nx < Sv, one, zero)
                    ok = ok * jnp.where(ny >= zero, one, zero)
                    ok = ok * jnp.where(ny < Sv, one, zero)
                    ok = ok * jnp.where(nz >= zero, one, zero)
                    ok = ok * jnp.where(nz < Sv, one, zero)
                else:
                    ax, ay, az = x - ox, y - oy, z - oz
                    nx, ny, nz = ax >> 1, ay >> 1, az >> 1
                    ok = jnp.where(ax >= zero, one, zero)
                    ok = ok * jnp.where((ax & one) == zero, one, zero)
                    ok = ok * jnp.where(nx < Sv, one, zero)
                    ok = ok * jnp.where(ay >= zero, one, zero)
                    ok = ok * jnp.where((ay & one) == zero, one, zero)
                    ok = ok * jnp.where(ny < Sv, one, zero)
                    ok = ok * jnp.where(az >= zero, one, zero)
                    ok = ok * jnp.where((az & one) == zero, one, zero)
                    ok = ok * jnp.where(nz < Sv, one, zero)
                key = (nx * Sv + ny) * Sv + nz
                key = jnp.where(ok > zero, key, jnp.full((16,), DUMP, jnp.int32))
                kb[pl.ds(i * 16, 16)] = key
                ob[pl.ds(i * 16, 16)] = ok
                return 0
            lax.fori_loop(0, nv, qstep, 0)
            pltpu.async_copy(grid.at[kb], gb, sem).wait()

            if mode == "x":
                def fx(i, _):
                    g = gb[pl.ds(i * 16, 16)]
                    ok = ob[pl.ds(i * 16, 16)] > 0
                    gb[pl.ds(i * 16, 16)] = jnp.where(ok, g, jnp.full((16,), SENT, jnp.int32))
                    return 0
                lax.fori_loop(0, nv, fx, 0)
                pltpu.sync_copy(gb.at[pl.ds(0, CH)], outs[0].at[pl.ds(k * Np + rbase, CH)])
            elif mode == "z":
                def fz(i, _, k=k):
                    g = gb[pl.ds(i * 16, 16)]
                    ok = ob[pl.ds(i * 16, 16)] > 0
                    g = jnp.where(ok, g, jnp.full((16,), SENT, jnp.int32))
                    gb[pl.ds(i * 16, 16)] = g * 27 + k
                    return 0
                lax.fori_loop(0, nv, fz, 0)
                pltpu.sync_copy(gb.at[pl.ds(0, CH)], outs[0].at[pl.ds(k * Np + rbase, CH)])
            else:
                def fb(i, _):
                    g = gb[pl.ds(i * 16, 16)]
                    ok = ob[pl.ds(i * 16, 16)] > 0
                    g = jnp.where(ok, g, jnp.full((16,), SENT, jnp.int32))
                    gb[pl.ds(i * 16, 16)] = g
                    kb[pl.ds(i * 16, 16)] = g * 27 + (26 - k)
                    return 0
                lax.fori_loop(0, nv, fb, 0)
                pltpu.sync_copy(gb.at[pl.ds(0, CH)], outs[0].at[pl.ds(k * Np + rbase, CH)])
                pltpu.sync_copy(kb.at[pl.ds(0, CH)], outs[1].at[pl.ds((26 - k) * Np + rbase, CH)])

    return body


def _kq(px, py, pz, grid, Np, CH, S, sh, stride, transposed, mode, SENT):
    nout = 2 if mode == "both" else 1
    outs = [jax.ShapeDtypeStruct((27 * Np,), jnp.int32) for _ in range(nout)]
    scratch = [pltpu.VMEM((CH,), jnp.int32) for _ in range(6)] + [pltpu.SemaphoreType.DMA]
    body = _make_kq(Np, CH, S, sh, stride, transposed, mode, SENT)
    return functools.partial(
        pl.kernel, out_type=tuple(outs) if nout > 1 else outs[0],
        mesh=_mesh(2), scratch_types=scratch, compiler_params=_SCP,
    )(body)(px, py, pz, grid)


def _make_kx(Np, CH, ci):
    def body(idx, ftab, xout, iv, rows, sem):
        wid = _wid2()
        rbase = wid * CH
        for k in range(27):
            pltpu.sync_copy(idx.at[pl.ds(k * Np + rbase, CH)], iv)
            pltpu.async_copy(ftab.at[iv], rows, sem).wait()
            pltpu.sync_copy(rows, xout.at[k, pl.ds(rbase, CH)])
    return body


def _kx(idx, ftab, Np, CH, ci):
    scratch = [pltpu.VMEM((CH,), jnp.int32), pltpu.VMEM((CH, ci), jnp.float32),
               pltpu.SemaphoreType.DMA]
    return functools.partial(
        pl.kernel, out_type=jax.ShapeDtypeStruct((27, Np + Bm, ci), jnp.float32),
        mesh=_mesh(2), scratch_types=scratch, compiler_params=_SCP,
    )(_make_kx(Np, CH, ci))(idx, ftab)


def _make_kz(Np, CH, co, chunk):
    nv = chunk * co // 16

    def body(zidx, ztab, yout, iv, acc, buf, sem):
        wid = _wid2()
        rbase = wid * CH

        def onechunk(c, _):
            start = rbase + c * chunk
            pltpu.sync_copy(zidx.at[pl.ds(0 * Np + start, chunk)], iv)
            pltpu.async_copy(ztab.at[iv], acc, sem).wait()
            for k in range(1, 27):
                pltpu.sync_copy(zidx.at[pl.ds(k * Np + start, chunk)], iv)
                pltpu.async_copy(ztab.at[iv], buf, sem).wait()

                def addrow(r, _):
                    for h in range(co // 16):
                        acc[r, pl.ds(h * 16, 16)] = acc[r, pl.ds(h * 16, 16)] + buf[r, pl.ds(h * 16, 16)]
                    return 0
                lax.fori_loop(0, chunk, addrow, 0)
            pltpu.sync_copy(acc, yout.at[pl.ds(start, chunk)])
            return 0
        lax.fori_loop(0, CH // chunk, onechunk, 0)
    return body


def _kz(zidx, ztab, Np, CH, co, chunk):
    scratch = [pltpu.VMEM((chunk,), jnp.int32), pltpu.VMEM((chunk, co), jnp.float32),
               pltpu.VMEM((chunk, co), jnp.float32), pltpu.SemaphoreType.DMA]
    return functools.partial(
        pl.kernel, out_type=jax.ShapeDtypeStruct((Np + Bm, co), jnp.float32),
        mesh=_mesh(2), scratch_types=scratch, compiler_params=_SCP,
    )(_make_kz(Np, CH, co, chunk))(zidx, ztab)


def _km_x(X, W, m, Np, ci, co):
    NB = Np // Bm

    def body(x_ref, w_ref, m_ref, y_ref, st_ref):
        i = pl.program_id(0)
        k = pl.program_id(1)

        @pl.when((i == 0) & (k == 0))
        def _():
            st_ref[...] = jnp.zeros_like(st_ref)

        @pl.when(i == NB)
        def _():
            y_ref[...] = jnp.zeros_like(y_ref)

        @pl.when(i < NB)
        def _():
            y = jnp.dot(x_ref[0], w_ref[0], preferred_element_type=jnp.float32)

            @pl.when(k == 0)
            def _():
                y_ref[...] = y

            @pl.when(k > 0)
            def _():
                y_ref[...] = y_ref[...] + y

            @pl.when(k == 26)
            def _():
                mf = m_ref[...].astype(jnp.float32)
                yy = y_ref[...]
                st_ref[0:1, :] = st_ref[0:1, :] + jnp.sum(yy * mf, axis=0, keepdims=True)
                st_ref[1:2, :] = st_ref[1:2, :] + jnp.sum(yy * yy * mf, axis=0, keepdims=True)
                st_ref[2:3, :] = st_ref[2:3, :] + jnp.sum(mf) * jnp.ones((1, co), jnp.float32)

    return pl.pallas_call(
        body,
        grid=(NB + 1, 27),
        in_specs=[
            pl.BlockSpec((1, Bm, ci), lambda i, k: (k, i, 0)),
            pl.BlockSpec((1, ci, co), lambda i, k: (k, 0, 0)),
            pl.BlockSpec((Bm, 1), lambda i, k: (i, 0)),
        ],
        out_specs=[
            pl.BlockSpec((Bm, co), lambda i, k: (i, 0)),
            pl.BlockSpec((8, co), lambda i, k: (0, 0)),
        ],
        out_shape=[
            jax.ShapeDtypeStruct((Np + Bm, co), jnp.float32),
            jax.ShapeDtypeStruct((8, co), jnp.float32),
        ],
    )(X, W, m)


def _km_z(d, Ws, Np, ci, KC):
    NB = Np // Bm

    def body(d_ref, w_ref, z_ref):
        z_ref[...] = jnp.dot(d_ref[...], w_ref[...], preferred_element_type=jnp.float32)

    return pl.pallas_call(
        body,
        grid=(NB + 1,),
        in_specs=[
            pl.BlockSpec((Bm, ci), lambda i: (i, 0)),
            pl.BlockSpec((ci, KC), lambda i: (0, 0)),
        ],
        out_specs=pl.BlockSpec((Bm, KC), lambda i: (i, 0)),
        out_shape=jax.ShapeDtypeStruct((Np + Bm, KC), jnp.float32),
    )(d, Ws)


def _ks(Y, m, Np, co):
    NB = Np // Bm

    def body(y_ref, m_ref, st_ref):
        i = pl.program_id(0)

        @pl.when(i == 0)
        def _():
            st_ref[...] = jnp.zeros_like(st_ref)

        mf = m_ref[...].astype(jnp.float32)
        yy = y_ref[...]
        st_ref[0:1, :] = st_ref[0:1, :] + jnp.sum(yy * mf, axis=0, keepdims=True)
        st_ref[1:2, :] = st_ref[1:2, :] + jnp.sum(yy * yy * mf, axis=0, keepdims=True)
        st_ref[2:3, :] = st_ref[2:3, :] + jnp.sum(mf) * jnp.ones((1, co), jnp.float32)

    return pl.pallas_call(
        body,
        grid=(NB,),
        in_specs=[
            pl.BlockSpec((Bm, co), lambda i: (i, 0)),
            pl.BlockSpec((Bm, 1), lambda i: (i, 0)),
        ],
        out_specs=pl.BlockSpec((8, co), lambda i: (0, 0)),
        out_shape=jax.ShapeDtypeStruct((8, co), jnp.float32),
    )(Y, m)


def _kb(Y, st, g, b, Np, co, keep, skip=None, cs=0):
    NB = Np // Bm
    width = keep + cs

    def body(*refs):
        if skip is not None:
            y_ref, st_ref, g_ref, b_ref, s_ref, o_ref = refs
        else:
            y_ref, st_ref, g_ref, b_ref, o_ref = refs
        i = pl.program_id(0)

        @pl.when(i == NB)
        def _():
            o_ref[...] = jnp.zeros_like(o_ref)

        @pl.when(i < NB)
        def _():
            s1 = st_ref[0:1, :]
            s2 = st_ref[1:2, :]
            n = st_ref[2:3, :]
            mu = s1 / n
            var = s2 / n - mu * mu
            sc = g_ref[...] / jnp.sqrt(var + 1e-5)
            sh = b_ref[...] - mu * sc
            yb = jnp.maximum(y_ref[...] * sc + sh, 0.0)
            if skip is not None:
                o_ref[...] = jnp.concatenate([yb[:, :keep], s_ref[...]], axis=1)
            else:
                o_ref[...] = yb[:, :keep]

    in_specs = [
        pl.BlockSpec((Bm, co), lambda i: (i, 0)),
        pl.BlockSpec((8, co), lambda i: (0, 0)),
        pl.BlockSpec((1, co), lambda i: (0, 0)),
        pl.BlockSpec((1, co), lambda i: (0, 0)),
    ]
    args = [Y, st, g, b]
    if skip is not None:
        in_specs.append(pl.BlockSpec((Bm, cs), lambda i: (i, 0)))
        args.append(skip)
    return pl.pallas_call(
        body,
        grid=(NB + 1,),
        in_specs=in_specs,
        out_specs=pl.BlockSpec((Bm, width), lambda i: (i, 0)),
        out_shape=jax.ShapeDtypeStruct((Np + Bm, width), jnp.float32),
    )(*args)


def kernel(coords, feats, params):
    N = coords.shape[0]
    CH = -(-N // (NW * 16)) * 16
    while (NW * CH) % Bm:
        CH += 16
    Np = NW * CH
    CH16 = Np // NT1
    SENT = Np
    p = params

    x = coords[:, 1].astype(jnp.int32)
    y = coords[:, 2].astype(jnp.int32)
    z = coords[:, 3].astype(jnp.int32)
    px = jnp.pad(x, (0, Np - N), constant_values=256)
    py = jnp.pad(y, (0, Np - N), constant_values=0)
    pz = jnp.pad(z, (0, Np - N), constant_values=0)

    grid1, grid2 = _build_grids(px, py, pz, Np, CH16, [(256, 0), (128, 1)])
    rep2 = _rep(px, py, pz, grid2, Np, CH, 128, 1, N)
    grid4 = _build_grids(px, py, pz, Np, CH16, [(64, 2)], masked_arr=rep2)
    rep4 = _rep(px, py, pz, grid4, Np, CH, 64, 2, N, parent=rep2)
    grid8 = _build_grids(px, py, pz, Np, CH16, [(32, 3)], masked_arr=rep4)
    rep8 = _rep(px, py, pz, grid8, Np, CH, 32, 3, N, parent=rep4)

    idxA, zidx0 = _kq(px, py, pz, grid1, Np, CH, 256, 0, 1, False, "both", SENT)
    idx1 = _kq(px, py, pz, grid1, Np, CH, 256, 1, 2, False, "x", SENT)
    idx2 = _kq(px, py, pz, grid2, Np, CH, 128, 2, 2, False, "x", SENT)
    idx3 = _kq(px, py, pz, grid4, Np, CH, 64, 3, 2, False, "x", SENT)
    zidx3 = _kq(px, py, pz, grid8, Np, CH, 32, 2, 2, True, "z", SENT)
    zidx2 = _kq(px, py, pz, grid4, Np, CH, 64, 1, 2, True, "z", SENT)
    zidx1 = _kq(px, py, pz, grid2, Np, CH, 128, 0, 2, True, "z", SENT)

    m1 = (jnp.arange(Np + Bm, dtype=jnp.int32) < N).astype(jnp.int32)[:, None]
    m2 = rep2[:, None]
    m4 = rep4[:, None]
    m8 = rep8[:, None]

    def g2(v, co):
        return jnp.pad(v, (0, co - v.shape[0]))[None, :]

    f0 = jnp.pad(feats, ((0, Np + Bm - N), (0, 4)))

    def enc(idx, ftab, W, g, b, m, ci, co):
        X = _kx(idx, ftab, Np, CH, ci)
        Yst = _km_x(X, W, m, Np, ci, co)
        return _kb(Yst[0], Yst[1], g2(g, co), g2(b, co), Np, co, co)

    W0p = jnp.pad(p['W0'], ((0, 0), (0, 4), (0, 0)))
    f1 = enc(idxA, f0, W0p, p['g0'], p['b0'], m1, 8, 8)
    f2 = enc(idx1, f1, p['W1'], p['g1'], p['b1'], m2, 8, 16)
    f4 = enc(idx2, f2, p['W2'], p['g2'], p['b2'], m4, 16, 32)
    f8 = enc(idx3, f4, p['W3'], p['g3'], p['b3'], m8, 32, 64)

    def wstack(W, co_pad):
        ci, co = W.shape[1], W.shape[2]
        Wp = jnp.pad(W, ((0, 0), (0, 0), (0, co_pad - co)))
        return Wp.transpose(1, 0, 2).reshape(ci, 27 * co_pad)

    def dec(zidx, d, W, g, b, m, co_pad, keep, skip=None, cs=0, chunk=784):
        ci = d.shape[1]
        Z = _km_z(d, wstack(W, co_pad), Np, ci, 27 * co_pad)
        Zr = Z.reshape((Np + Bm) * 27, co_pad)
        Y = _kz(zidx, Zr, Np, CH, co_pad, chunk)
        if g is None:
            return Y
        st = _ks(Y, m, Np, co_pad)
        return _kb(Y, st, g2(g, co_pad), g2(b, co_pad), Np, co_pad, keep, skip, cs)

    d = dec(zidx3, f8, p['Wt3'], p['g4'], p['b4'], m4, 32, 32, skip=f4, cs=32)
    d = dec(zidx2, d, p['Wt2'], p['g5'], p['b5'], m2, 32, 32, skip=f2, cs=16)
    d = dec(zidx1, d, p['Wt1'], p['g6'], p['b6'], m1, 32, 24, skip=f1, cs=8)
    out = dec(zidx0, d, p['Wt0'], None, None, None, 16, 16, chunk=1568)
    return out[:N, :2]
